# Initial kernel scaffold; baseline (speedup 1.0000x reference)
import functools

import jax
import jax.numpy as jnp
from jax import lax
from jax.experimental import pallas as pl
from jax.experimental.pallas import tpu as pltpu
from jax.experimental.pallas import tpu_sc as plsc

_NUM_FIELDS = 26
_VOCAB = 100000
_EMBED_DIM = 32
_BATCH = 16384

_ROWS = _BATCH * _NUM_FIELDS
_NW = 32
_ROWS_W = _ROWS // _NW
_CHUNK = 1664
_NCHUNK = _ROWS_W // _CHUNK
_G = _CHUNK // 128

_mesh = plsc.VectorSubcoreMesh(core_axis_name="c", subcore_axis_name="s")


@functools.partial(
    pl.kernel,
    mesh=_mesh,
    out_type=jax.ShapeDtypeStruct((_ROWS, _EMBED_DIM), jnp.float32),
    scratch_types=[
        pltpu.VMEM((_G, 128), jnp.int32),
        pltpu.VMEM((_G, 128), jnp.int32),
        pltpu.VMEM((_G, 128), jnp.int32),
        pltpu.VMEM((_CHUNK, _EMBED_DIM), jnp.float32),
        pltpu.SemaphoreType.DMA,
    ],
)
def _embed_gather(x_hbm, offs_hbm, table_hbm, out_hbm, xv, idxv, offsv, rows, sem):
    wid = lax.axis_index("s") * 2 + lax.axis_index("c")
    pltpu.sync_copy(offs_hbm, offsv)

    def chunk_body(c, carry):
        row0 = wid * (_ROWS_W // 128) + c * _G
        pltpu.sync_copy(x_hbm.at[pl.ds(row0, _G)], xv)
        for j in range(_G):
            for k in range(128 // 16):
                sl = pl.ds(k * 16, 16)
                idxv[j, sl] = xv[j, sl] + offsv[j, sl]
        copies = [
            pltpu.async_copy(table_hbm.at[idxv.at[j]],
                             rows.at[pl.ds(j * 128, 128)], sem)
            for j in range(_G)
        ]
        for cp in copies:
            cp.wait()
        base = wid * _ROWS_W + c * _CHUNK
        pltpu.sync_copy(rows, out_hbm.at[pl.ds(base, _CHUNK)])
        return carry

    lax.fori_loop(0, _NCHUNK, chunk_body, 0)


@jax.jit
def kernel(x, tables):
    table_flat = tables.reshape(_NUM_FIELDS * _VOCAB, _EMBED_DIM)
    x_flat = x.reshape(_ROWS // 128, 128)
    offs = jnp.tile(
        jnp.arange(_NUM_FIELDS, dtype=jnp.int32) * _VOCAB, _CHUNK // _NUM_FIELDS
    ).reshape(_G, 128)
    out = _embed_gather(x_flat, offs, table_flat)
    return out.reshape(_BATCH, _NUM_FIELDS * _EMBED_DIM)


# speedup vs baseline: 1.2075x; 1.2075x over previous
import functools

import jax
import jax.numpy as jnp
from jax import lax
from jax.experimental import pallas as pl
from jax.experimental.pallas import tpu as pltpu
from jax.experimental.pallas import tpu_sc as plsc

_NUM_FIELDS = 26
_VOCAB = 100000
_EMBED_DIM = 32
_BATCH = 16384

_ROWS = _BATCH * _NUM_FIELDS
_NW = 32
_ROWS_W = _ROWS // _NW
_CHUNK = 1664
_NCHUNK = _ROWS_W // _CHUNK
_G = _CHUNK // 128

_mesh = plsc.VectorSubcoreMesh(core_axis_name="c", subcore_axis_name="s")


@functools.partial(
    pl.kernel,
    mesh=_mesh,
    out_type=jax.ShapeDtypeStruct((_ROWS, _EMBED_DIM), jnp.float32),
    compiler_params=pltpu.CompilerParams(use_tc_tiling_on_sc=False),
    scratch_types=[
        pltpu.VMEM((_CHUNK,), jnp.int32),
        pltpu.VMEM((_G, 128), jnp.int32),
        pltpu.VMEM((_CHUNK,), jnp.int32),
        pltpu.VMEM((_CHUNK, _EMBED_DIM), jnp.float32),
        pltpu.SemaphoreType.DMA,
    ],
)
def _embed_gather(x_hbm, offs_hbm, table_hbm, out_hbm, xv, idxv, offsv, rows, sem):
    wid = lax.axis_index("s") * 2 + lax.axis_index("c")
    pltpu.sync_copy(offs_hbm, offsv)

    def chunk_body(c, carry):
        base = wid * _ROWS_W + c * _CHUNK
        pltpu.sync_copy(x_hbm.at[pl.ds(base, _CHUNK)], xv)
        for j in range(_G):
            for k in range(128 // 16):
                p = j * 128 + k * 16
                idxv[j, pl.ds(k * 16, 16)] = xv[pl.ds(p, 16)] + offsv[pl.ds(p, 16)]
        copies = [
            pltpu.async_copy(table_hbm.at[idxv.at[j]],
                             rows.at[pl.ds(j * 128, 128)], sem)
            for j in range(_G)
        ]
        for cp in copies:
            cp.wait()
        pltpu.sync_copy(rows, out_hbm.at[pl.ds(base, _CHUNK)])
        return carry

    lax.fori_loop(0, _NCHUNK, chunk_body, 0)


@jax.jit
def kernel(x, tables):
    table_flat = tables.reshape(_NUM_FIELDS * _VOCAB, _EMBED_DIM)
    x_flat = x.reshape(_ROWS)
    offs = jnp.tile(
        jnp.arange(_NUM_FIELDS, dtype=jnp.int32) * _VOCAB, _CHUNK // _NUM_FIELDS
    )
    out = _embed_gather(x_flat, offs, table_flat)
    return out.reshape(_BATCH, _NUM_FIELDS * _EMBED_DIM)


# speedup vs baseline: 1.2099x; 1.0020x over previous
import functools

import jax
import jax.numpy as jnp
from jax import lax
from jax.experimental import pallas as pl
from jax.experimental.pallas import tpu as pltpu
from jax.experimental.pallas import tpu_sc as plsc

_NUM_FIELDS = 26
_VOCAB = 100000
_EMBED_DIM = 32
_BATCH = 16384

_ROWS = _BATCH * _NUM_FIELDS
_NW = 32
_ROWS_W = _ROWS // _NW
_CHUNK = 1664
_NCHUNK = _ROWS_W // _CHUNK
_G = _CHUNK // 128

_mesh = plsc.VectorSubcoreMesh(core_axis_name="c", subcore_axis_name="s")


@functools.partial(
    pl.kernel,
    mesh=_mesh,
    out_type=jax.ShapeDtypeStruct((_ROWS, _EMBED_DIM), jnp.float32),
    compiler_params=pltpu.CompilerParams(use_tc_tiling_on_sc=False),
    scratch_types=[
        pltpu.VMEM((2, _CHUNK), jnp.int32),
        pltpu.VMEM((2, _G, 128), jnp.int32),
        pltpu.VMEM((_CHUNK,), jnp.int32),
        pltpu.VMEM((2, _CHUNK, _EMBED_DIM), jnp.float32),
        pltpu.SemaphoreType.DMA,
        pltpu.SemaphoreType.DMA,
        pltpu.SemaphoreType.DMA,
        pltpu.SemaphoreType.DMA,
    ],
)
def _embed_gather(x_hbm, offs_hbm, table_hbm, out_hbm,
                  xv, idxv, offsv, rows, g0, g1, w0, w1):
    wid = lax.axis_index("s") * 2 + lax.axis_index("c")
    gsem = (g0, g1)
    wsem = (w0, w1)
    pltpu.sync_copy(offs_hbm, offsv)

    def stage_and_fire(c, b):
        base = wid * _ROWS_W + c * _CHUNK
        pltpu.sync_copy(x_hbm.at[pl.ds(base, _CHUNK)], xv.at[b])
        for j in range(_G):
            for k in range(128 // 16):
                p = j * 128 + k * 16
                idxv[b, j, pl.ds(k * 16, 16)] = (
                    xv[b, pl.ds(p, 16)] + offsv[pl.ds(p, 16)])
        return [
            pltpu.async_copy(table_hbm.at[idxv.at[b, j]],
                             rows.at[b, pl.ds(j * 128, 128)], gsem[b])
            for j in range(_G)
        ]

    def writeback(c, b):
        base = wid * _ROWS_W + c * _CHUNK
        return pltpu.async_copy(rows.at[b], out_hbm.at[pl.ds(base, _CHUNK)],
                                wsem[b])

    gathers = [None, None]
    writes = [None, None]
    gathers[0] = stage_and_fire(0, 0)
    for c in range(_NCHUNK):
        b = c & 1
        nb = b ^ 1
        if c + 1 < _NCHUNK:
            if writes[nb] is not None:
                writes[nb].wait()
                writes[nb] = None
            gathers[nb] = stage_and_fire(c + 1, nb)
        for cp in gathers[b]:
            cp.wait()
        writes[b] = writeback(c, b)
    for b in range(2):
        if writes[b] is not None:
            writes[b].wait()


@jax.jit
def kernel(x, tables):
    table_flat = tables.reshape(_NUM_FIELDS * _VOCAB, _EMBED_DIM)
    x_flat = x.reshape(_ROWS)
    offs = jnp.tile(
        jnp.arange(_NUM_FIELDS, dtype=jnp.int32) * _VOCAB, _CHUNK // _NUM_FIELDS
    )
    out = _embed_gather(x_flat, offs, table_flat)
    return out.reshape(_BATCH, _NUM_FIELDS * _EMBED_DIM)


# speedup vs baseline: 1.7721x; 1.4646x over previous
import functools
import jax
import jax.numpy as jnp
from jax import lax
from jax.experimental import pallas as pl
from jax.experimental.pallas import tpu as pltpu
from jax.experimental.pallas import tpu_sc as plsc

_F = 26
_V = 100000
_D = 32
_B = 16384
_NW = 32
_NT = _F * _D
_BH = _B // 2

_mesh = plsc.VectorSubcoreMesh(core_axis_name="c", subcore_axis_name="s")


@functools.partial(
    pl.kernel,
    mesh=_mesh,
    out_type=jax.ShapeDtypeStruct((_NT, _B), jnp.float32),
    compiler_params=pltpu.CompilerParams(
        use_tc_tiling_on_sc=False, needs_layout_passes=False),
    scratch_types=[
        pltpu.VMEM((_V,), jnp.float32),
        pltpu.VMEM((_BH,), jnp.int32),
        pltpu.VMEM((_BH,), jnp.float32),
        pltpu.SemaphoreType.DMA,
    ],
)
def _embed_cols(xt_hbm, tab_hbm, out_hbm, slicev, xfv, colv, sem):
    wid = lax.axis_index("s") * 2 + lax.axis_index("c")

    def task_body(t, carry):
        c = t * _NW + wid
        f = c // _D
        d = c - f * _D
        pltpu.sync_copy(tab_hbm.at[f, d], slicev)
        def half_body(h, carry2):
            pltpu.sync_copy(xt_hbm.at[f, pl.ds(h * _BH, _BH)], xfv)
            def vec_body(k, carry3):
                sl = pl.ds(k * 16, 16)
                colv[sl] = plsc.load_gather(slicev, [xfv[sl]])
                return carry3
            lax.fori_loop(0, _BH // 16, vec_body, 0)
            pltpu.sync_copy(colv, out_hbm.at[c, pl.ds(h * _BH, _BH)])
            return carry2
        lax.fori_loop(0, 2, half_body, 0)
        return carry

    lax.fori_loop(0, _NT // _NW, task_body, 0)


@jax.jit
def kernel(x, tables):
    xt = jnp.transpose(x)
    tt = jnp.transpose(tables, (0, 2, 1))
    out_t = _embed_cols(xt, tt)
    return jnp.transpose(out_t)


# speedup vs baseline: 1.8781x; 1.0598x over previous
import functools
import jax
import jax.numpy as jnp
from jax import lax
from jax.experimental import pallas as pl
from jax.experimental.pallas import tpu as pltpu
from jax.experimental.pallas import tpu_sc as plsc

_F = 26
_V = 100000
_D = 32
_B = 16384
_NW = 32
_NT = _F * _D
_BH = _B // 2

_mesh = plsc.VectorSubcoreMesh(core_axis_name="c", subcore_axis_name="s")


@functools.partial(
    pl.kernel,
    mesh=_mesh,
    out_type=jax.ShapeDtypeStruct((_NT, _B), jnp.float32),
    compiler_params=pltpu.CompilerParams(
        use_tc_tiling_on_sc=False, needs_layout_passes=False),
    scratch_types=[
        pltpu.VMEM((_V,), jnp.float32),
        pltpu.VMEM((_BH,), jnp.int32),
        pltpu.VMEM((_BH,), jnp.float32),
        pltpu.SemaphoreType.DMA,
    ],
)
def _embed_cols(xt_hbm, tab_hbm, out_hbm, slicev, xfv, colv, sem):
    wid = lax.axis_index("s") * 2 + lax.axis_index("c")

    def task_body(t, carry):
        c = t * _NW + wid
        f = c // _D
        d = c - f * _D
        pltpu.sync_copy(tab_hbm.at[f, d], slicev)
        def half_body(h, carry2):
            pltpu.sync_copy(xt_hbm.at[f, pl.ds(h * _BH, _BH)], xfv)
            def vec_body(k, carry3):
                for u in range(8):
                    sl = pl.ds(k * 128 + u * 16, 16)
                    colv[sl] = plsc.load_gather(slicev, [xfv[sl]])
                return carry3
            lax.fori_loop(0, _BH // 128, vec_body, 0)
            pltpu.sync_copy(colv, out_hbm.at[c, pl.ds(h * _BH, _BH)])
            return carry2
        lax.fori_loop(0, 2, half_body, 0)
        return carry

    lax.fori_loop(0, _NT // _NW, task_body, 0)


@jax.jit
def kernel(x, tables):
    xt = jnp.transpose(x)
    tt = jnp.transpose(tables, (0, 2, 1))
    out_t = _embed_cols(xt, tt)
    return jnp.transpose(out_t)
